# baseline (speedup 1.0000x reference)
import functools

import jax
import jax.numpy as jnp
from jax import lax
from jax.experimental import pallas as pl
from jax.experimental.pallas import tpu as pltpu
from jax.experimental.pallas import tpu_sc as plsc

HIDDEN = 64
BATCH = 16384
NCOLS = 1000001
CW = 512
NCHUNKS = 1953
TAIL = NCHUNKS * CW
NRING = 4
OUTROWS = BATCH + 16


def kernel(labels, embedding_table):
    info = plsc.get_sparse_core_info()
    nc, ns = info.num_cores, info.num_subcores
    nw = nc * ns
    cpw = (NCHUNKS + nw - 1) // nw

    mesh = plsc.VectorSubcoreMesh(core_axis_name="c", subcore_axis_name="s")

    @functools.partial(
        pl.kernel,
        mesh=mesh,
        out_type=jax.ShapeDtypeStruct((OUTROWS, 128), jnp.float32),
        scratch_types=[
            pltpu.VMEM((BATCH,), jnp.int32),
            pltpu.VMEM((BATCH + 16,), jnp.int32),
            pltpu.VMEM((HIDDEN, CW), jnp.float32),
            pltpu.VMEM((NRING, 16, 128), jnp.float32),
            pltpu.SemaphoreType.DMA,
            pltpu.SemaphoreType.DMA,
        ],
        compiler_params=pltpu.CompilerParams(
            use_tc_tiling_on_sc=True, needs_layout_passes=False
        ),
    )
    def emb(labels_hbm, table_t_hbm, out_hbm, labv, posq, stage, rowbuf, sem, rsem):
        sid = lax.axis_index("s")
        wid = sid * nc + lax.axis_index("c")
        g0 = wid * cpw
        clo = g0 * CW
        chi = jnp.minimum(clo + cpw * CW, TAIL)
        pltpu.sync_copy(labels_hbm, labv)

        lanes = lax.iota(jnp.int32, 16)

        def prescan(i, off):
            v = labv[pl.ds(i * 16, 16)]
            m = (v >= clo) & (v < chi)
            plsc.store_compressed(posq.at[pl.ds(off, 16)], lanes + i * 16, mask=m)
            cnt = jnp.max(plsc.all_reduce_population_count(m))
            return off + cnt

        nq = lax.fori_loop(0, BATCH // 16, prescan, jnp.int32(0))
        nqv = (nq + 15) // 16

        def chunk(k, fired):
            g = g0 + k
            c0 = g * CW

            @pl.when(g < NCHUNKS)
            def _():
                copies = []
                for i in range(8):
                    copies.append(
                        pltpu.async_copy(
                            table_t_hbm.at[pl.ds(8 * i, 8), pl.ds(c0, CW)],
                            stage.at[pl.ds(8 * i, 8), :],
                            sem,
                        )
                    )
                for c in copies:
                    c.wait()

            def qscan(j, f):
                pos = jnp.clip(posq[pl.ds(j * 16, 16)], 0, BATCH - 1)
                lab = plsc.load_gather(labv, [pos])
                valid = (lanes + j * 16) < nq
                m = valid & (lab >= c0) & (lab < c0 + CW)
                nhit = jnp.max(plsc.all_reduce_population_count(m))

                def fire(f):
                    slot = f % NRING

                    @pl.when(f >= NRING)
                    def _():
                        pltpu.make_async_copy(
                            out_hbm.at[pl.ds(0, 16), :], rowbuf.at[0], rsem
                        ).wait()

                    cols = jnp.clip(lab - c0, 0, CW - 1)
                    slot16 = jnp.full((16,), slot, jnp.int32)
                    for h in range(HIDDEN):
                        h16 = jnp.full((16,), h, jnp.int32)
                        vals = plsc.load_gather(stage, [h16, cols])
                        plsc.store_scatter(rowbuf, [slot16, lanes, h16], vals)
                    spos = jnp.where(m, pos, BATCH + lanes)
                    pltpu.async_copy(rowbuf.at[slot], out_hbm.at[spos], rsem)
                    return f + 1

                return lax.cond(nhit > 0, fire, lambda f: f, f)

            inner = lax.cond(
                g < NCHUNKS,
                lambda f: lax.fori_loop(0, nqv, qscan, f),
                lambda f: f,
                fired,
            )
            return inner

        nfired = lax.fori_loop(0, cpw, chunk, jnp.int32(0))

        def drain(i, c):
            pltpu.make_async_copy(
                out_hbm.at[pl.ds(0, 16), :], rowbuf.at[0], rsem
            ).wait()
            return c

        lax.fori_loop(0, jnp.minimum(nfired, NRING), drain, jnp.int32(0))

    out = emb(labels, embedding_table.T)
    core = out[:BATCH, :HIDDEN]
    tail_tab = embedding_table[TAIL:, :]
    tmask = labels >= TAIL
    fix = jnp.take(tail_tab, jnp.where(tmask, labels - TAIL, 0), axis=0)
    return jnp.where(tmask[:, None], fix, core)


# speedup vs baseline: 3.2539x; 3.2539x over previous
import functools

import jax
import jax.numpy as jnp
from jax import lax
from jax.experimental import pallas as pl
from jax.experimental.pallas import tpu as pltpu
from jax.experimental.pallas import tpu_sc as plsc

HIDDEN = 64
BATCH = 16384
NCOLS = 1000001
CW = 512
NCHUNKS = 1953
TAIL = NCHUNKS * CW
NRING = 4
OUTROWS = BATCH + 16
QCAP = BATCH + 16


def kernel(labels, embedding_table):
    info = plsc.get_sparse_core_info()
    nc, ns = info.num_cores, info.num_subcores
    nw = nc * ns
    cpw = (NCHUNKS + nw - 1) // nw
    nbits = max(1, (cpw - 1).bit_length())

    mesh = plsc.VectorSubcoreMesh(core_axis_name="c", subcore_axis_name="s")

    @functools.partial(
        pl.kernel,
        mesh=mesh,
        out_type=jax.ShapeDtypeStruct((OUTROWS, 128), jnp.float32),
        scratch_types=[
            pltpu.VMEM((QCAP,), jnp.int32),
            pltpu.VMEM((QCAP,), jnp.int32),
            pltpu.VMEM((2, HIDDEN, CW), jnp.float32),
            pltpu.VMEM((NRING, 16, 128), jnp.float32),
            pltpu.SemaphoreType.DMA,
            pltpu.SemaphoreType.DMA,
        ],
        compiler_params=pltpu.CompilerParams(
            use_tc_tiling_on_sc=True, needs_layout_passes=False
        ),
    )
    def emb(labels_hbm, table_t_hbm, out_hbm, xq, yq, stage, rowbuf, sem, rsem):
        sid = lax.axis_index("s")
        wid = sid * nc + lax.axis_index("c")
        g0 = wid * cpw
        clo = g0 * CW
        chi = jnp.minimum(clo + cpw * CW, TAIL)
        pltpu.sync_copy(labels_hbm, xq.at[pl.ds(0, BATCH)])

        lanes = lax.iota(jnp.int32, 16)

        def prescan(i, off):
            lab = xq[pl.ds(i * 16, 16)]
            m = (lab >= clo) & (lab < chi)
            rel = lab - clo
            key = ((rel >> 9) << 23) | ((rel & (CW - 1)) << 14) | (lanes + i * 16)
            plsc.store_compressed(yq.at[pl.ds(off, 16)], key, mask=m)
            return off + plsc.all_reduce_population_count(m)[0]

        nq = lax.fori_loop(0, BATCH // 16, prescan, jnp.int32(0))
        nqv = (nq + 15) // 16

        def radix_pass(src, dst, bit):
            def count(j, n):
                k = src[pl.ds(j * 16, 16)]
                valid = (lanes + j * 16) < nq
                mlow = valid & (((k >> (23 + bit)) & 1) == 0)
                return n + plsc.all_reduce_population_count(mlow)[0]

            nlow = lax.fori_loop(0, nqv, count, jnp.int32(0))

            def place(j, cur):
                lo, hi = cur
                k = src[pl.ds(j * 16, 16)]
                valid = (lanes + j * 16) < nq
                b = ((k >> (23 + bit)) & 1) == 1
                mlow = valid & (~b)
                mhigh = valid & b
                plsc.store_compressed(dst.at[pl.ds(lo, 16)], k, mask=mlow)
                plsc.store_compressed(dst.at[pl.ds(hi, 16)], k, mask=mhigh)
                return (
                    lo + plsc.all_reduce_population_count(mlow)[0],
                    hi + plsc.all_reduce_population_count(mhigh)[0],
                )

            lax.fori_loop(0, nqv, place, (jnp.int32(0), nlow))

        bufs = (yq, xq)
        for b in range(nbits):
            radix_pass(bufs[b % 2], bufs[(b + 1) % 2], b)
        q = bufs[nbits % 2]

        def fire_chunk(k):
            for i in range(8):
                pltpu.async_copy(
                    table_t_hbm.at[pl.ds(8 * i, 8), pl.ds((g0 + k) * CW, CW)],
                    stage.at[k % 2, pl.ds(8 * i, 8), :],
                    sem,
                )

        def drain_chunk():
            pltpu.make_async_copy(
                table_t_hbm.at[:, pl.ds(0, CW)], stage.at[0], sem
            ).wait()

        fire_chunk(0)

        def chunk(k, carry):
            def body(carry):
                ptr, f = carry

                @pl.when((k + 1 < cpw) & (g0 + k + 1 < NCHUNKS))
                def _():
                    fire_chunk(k + 1)

                drain_chunk()
                sl = k % 2

                def ext_cond(st):
                    return st[2]

                def ext_body(st):
                    ptr, f, _ = st
                    key = q[pl.ds(ptr, 16)]
                    valid = (ptr + lanes) < nq
                    m = valid & ((key >> 23) == k)
                    nm = plsc.all_reduce_population_count(m)[0]

                    def fire(f):
                        slot = f % NRING

                        @pl.when(f >= NRING)
                        def _():
                            pltpu.make_async_copy(
                                out_hbm.at[pl.ds(0, 16), :], rowbuf.at[0], rsem
                            ).wait()

                        cols = (key >> 14) & (CW - 1)
                        slot16 = jnp.full((16,), slot, jnp.int32)
                        for h in range(HIDDEN):
                            h16 = jnp.full((16,), h, jnp.int32)
                            vals = plsc.load_gather(stage.at[sl], [h16, cols])
                            plsc.store_scatter(rowbuf, [slot16, lanes, h16], vals)
                        spos = jnp.where(m, key & (BATCH - 1), BATCH + lanes)
                        pltpu.async_copy(rowbuf.at[slot], out_hbm.at[spos], rsem)
                        return f + 1

                    f2 = lax.cond(nm > 0, fire, lambda f: f, f)
                    return (ptr + nm, f2, nm == 16)

                ptr, f, _ = lax.while_loop(
                    ext_cond, ext_body, (ptr, f, jnp.bool_(True))
                )
                return (ptr, f)

            return lax.cond(g0 + k < NCHUNKS, body, lambda c: c, carry)

        _, nfired = lax.fori_loop(0, cpw, chunk, (jnp.int32(0), jnp.int32(0)))

        def drain_rows(i, c):
            pltpu.make_async_copy(
                out_hbm.at[pl.ds(0, 16), :], rowbuf.at[0], rsem
            ).wait()
            return c

        lax.fori_loop(0, jnp.minimum(nfired, NRING), drain_rows, jnp.int32(0))

    out = emb(labels, embedding_table.T)
    core = out[:BATCH, :HIDDEN]
    tail_tab = embedding_table[TAIL:, :]
    tmask = labels >= TAIL
    fix = jnp.take(tail_tab, jnp.where(tmask, labels - TAIL, 0), axis=0)
    return jnp.where(tmask[:, None], fix, core)
